# baseline (speedup 1.0000x reference)
import jax
import jax.numpy as jnp
from jax.experimental import pallas as pl
from jax.experimental.pallas import tpu as pltpu

_C = 19
_RB = 256
_B = _RB * 128
_PK = 128 * _C
_EPS = 1e-30


def _owloss_tc_kernel(g_ref, x_ref, mav_ref, pc_ref, out_ref,
                      wbig_ref, k_ref, m19_ref, iom_ref, acc_s, acc_c):
    i = pl.program_id(0)
    nsteps = pl.num_programs(0)

    @pl.when(i == 0)
    def _init():
        acc_s[...] = jnp.zeros_like(acc_s)
        acc_c[...] = jnp.zeros_like(acc_c)
        mav = mav_ref[...]
        mns = jnp.sum(mav * mav, axis=1, keepdims=True)
        w = (mav * jax.lax.rsqrt(jnp.maximum(mns, _EPS))).astype(jnp.bfloat16)
        qi = jax.lax.broadcasted_iota(jnp.int32, (_PK, _C), 0)
        ci = jax.lax.broadcasted_iota(jnp.int32, (_PK, _C), 1)
        m19 = (qi % _C == ci).astype(jnp.bfloat16)
        m19_ref[...] = m19
        qj = jax.lax.broadcasted_iota(jnp.int32, (_PK, 128), 0)
        jj = jax.lax.broadcasted_iota(jnp.int32, (_PK, 128), 1)
        k_ref[...] = (qj // _C == jj).astype(jnp.bfloat16)
        iom_ref[...] = (jax.lax.broadcasted_iota(jnp.int32, (8, _PK), 1)
                        % _C).astype(jnp.float32)
        a1 = jax.lax.dot_general(m19, w, (((1,), (1,)), ((), ())),
                                 preferred_element_type=jnp.float32)
        w0 = jax.lax.dot_general(a1.astype(jnp.bfloat16), m19,
                                 (((1,), (1,)), ((), ())),
                                 preferred_element_type=jnp.float32)
        bd = jax.lax.dot_general(k_ref[...], k_ref[...],
                                 (((1,), (1,)), ((), ())),
                                 preferred_element_type=jnp.float32)
        wbig_ref[...] = (w0 * bd).astype(jnp.bfloat16)

    xp = x_ref[...]
    xb = xp.astype(jnp.bfloat16)
    kk = k_ref[...]
    at = jax.lax.dot_general(xb, wbig_ref[...], (((1,), (0,)), ((), ())),
                             preferred_element_type=jnp.float32)
    g2 = g_ref[0]
    gb = g2.astype(jnp.bfloat16)
    g_p = jax.lax.dot_general(gb, kk, (((1,), (1,)), ((), ())),
                              preferred_element_type=jnp.float32)
    msk = g_p == iom_ref[0:1, :]
    mskb = msk.astype(jnp.bfloat16)
    x2 = xb * xb
    nsq = jax.lax.dot_general(x2, kk, (((1,), (0,)), ((), ())),
                              preferred_element_type=jnp.float32)
    rnl = jax.lax.rsqrt(jnp.maximum(nsq, _EPS))
    rnl_p = jax.lax.dot_general(rnl.astype(jnp.bfloat16), kk,
                                (((1,), (1,)), ((), ())),
                                preferred_element_type=jnp.float32)
    acc_s[...] += (at * rnl_p) * mskb.astype(jnp.float32)
    acc_c[...] += mskb.astype(jnp.float32)

    @pl.when(i == nsteps - 1)
    def _finish():
        m19f = m19_ref[...].astype(jnp.float32)
        srb = jax.lax.dot_general(acc_s[...], m19f, (((1,), (0,)), ((), ())),
                                  preferred_element_type=jnp.float32)
        crb = jax.lax.dot_general(acc_c[...], m19f, (((1,), (0,)), ((), ())),
                                  preferred_element_type=jnp.float32)
        cs = jnp.sum(srb, axis=0, keepdims=True)
        c = jnp.sum(crb, axis=0, keepdims=True)
        pc = pc_ref[...]
        lbl = jax.lax.broadcasted_iota(jnp.int32, (1, _C), 1)
        present = c > 0.0
        minl = jnp.min(jnp.where(present, lbl, _C))
        include = present & (lbl != minl) & (pc > 0.0)
        means = (c - cs) / jnp.maximum(c, 1.0)
        terms = jnp.where(include, means, 0.0)
        out_ref[...] = jnp.sum(terms, axis=(0, 1), keepdims=True).reshape(1, 1)


def kernel(logits, sem_gt, is_train, mav_table, prev_count):
    n = logits.shape[0]
    nsteps = n // _B
    g3 = sem_gt.reshape(nsteps, _RB, 128)
    xpk = logits.reshape(n // 128, _PK)
    pc2 = prev_count.reshape(1, _C)
    out = pl.pallas_call(
        _owloss_tc_kernel,
        grid=(nsteps,),
        in_specs=[
            pl.BlockSpec((1, _RB, 128), lambda i: (i, 0, 0)),
            pl.BlockSpec((_RB, _PK), lambda i: (i, 0)),
            pl.BlockSpec((_C, _C), lambda i: (0, 0)),
            pl.BlockSpec((1, _C), lambda i: (0, 0)),
        ],
        out_specs=pl.BlockSpec((1, 1), lambda i: (0, 0)),
        out_shape=jax.ShapeDtypeStruct((1, 1), jnp.float32),
        scratch_shapes=[
            pltpu.VMEM((_PK, _PK), jnp.bfloat16),
            pltpu.VMEM((_PK, 128), jnp.bfloat16),
            pltpu.VMEM((_PK, _C), jnp.bfloat16),
            pltpu.VMEM((8, _PK), jnp.float32),
            pltpu.VMEM((_RB, _PK), jnp.float32),
            pltpu.VMEM((_RB, _PK), jnp.float32),
        ],
        compiler_params=pltpu.CompilerParams(
            dimension_semantics=("arbitrary",),
        ),
    )(g3, xpk, mav_table, pc2)
    return jnp.reshape(out, ())


# speedup vs baseline: 6.8308x; 6.8308x over previous
import jax
import jax.numpy as jnp
from jax.experimental import pallas as pl
from jax.experimental.pallas import tpu as pltpu

_C = 19
_B = 32768
_EPS = 1e-30


def _owloss_tc_kernel(g_ref, x_ref, mav_ref, pc_ref, out_ref, wb_ref, acc):
    i = pl.program_id(0)
    nsteps = pl.num_programs(0)

    @pl.when(i == 0)
    def _init():
        acc[...] = jnp.zeros_like(acc)
        mav = mav_ref[...]
        mns = jnp.sum(mav * mav, axis=1, keepdims=True)
        w = mav * jax.lax.rsqrt(jnp.maximum(mns, _EPS))
        wb_ref[...] = w.astype(jnp.bfloat16)

    xt = x_ref[...]
    g = g_ref[0]

    at = jax.lax.dot_general(wb_ref[...], xt, (((1,), (0,)), ((), ())),
                             preferred_element_type=jnp.float32)
    ones_row = jnp.ones((1, _C), jnp.bfloat16)
    nsq = jax.lax.dot_general(ones_row, xt * xt, (((1,), (0,)), ((), ())),
                              preferred_element_type=jnp.float32)
    rnl = jax.lax.rsqrt(jnp.maximum(nsq, _EPS))

    lbl = jax.lax.broadcasted_iota(jnp.int32, (_C, 1), 0)
    msk = lbl == g
    num = jnp.sum(jnp.where(msk, at, 0.0), axis=0, keepdims=True)
    cos = (num * rnl).astype(jnp.bfloat16)
    ohb = msk.astype(jnp.bfloat16)
    cat = jnp.concatenate([cos, jnp.ones((1, _B), jnp.bfloat16)], axis=0)
    z = jax.lax.dot_general(ohb, cat, (((1,), (1,)), ((), ())),
                            preferred_element_type=jnp.float32)
    acc[...] += z

    @pl.when(i == nsteps - 1)
    def _finish():
        cs = acc[:, 0:1]
        c = acc[:, 1:2]
        pc = pc_ref[...]
        present = c > 0.0
        minl = jnp.min(jnp.where(present, lbl, _C))
        include = present & (lbl != minl) & (pc > 0.0)
        means = (c - cs) / jnp.maximum(c, 1.0)
        terms = jnp.where(include, means, 0.0)
        out_ref[...] = jnp.sum(terms, axis=(0, 1), keepdims=True).reshape(1, 1)


def kernel(logits, sem_gt, is_train, mav_table, prev_count):
    n = logits.shape[0]
    nsteps = n // _B
    xt = logits.astype(jnp.bfloat16).T
    g3 = sem_gt.reshape(nsteps, 1, _B)
    pc2 = prev_count.reshape(_C, 1)
    out = pl.pallas_call(
        _owloss_tc_kernel,
        grid=(nsteps,),
        in_specs=[
            pl.BlockSpec((1, 1, _B), lambda i: (i, 0, 0)),
            pl.BlockSpec((_C, _B), lambda i: (0, i)),
            pl.BlockSpec((_C, _C), lambda i: (0, 0)),
            pl.BlockSpec((_C, 1), lambda i: (0, 0)),
        ],
        out_specs=pl.BlockSpec((1, 1), lambda i: (0, 0)),
        out_shape=jax.ShapeDtypeStruct((1, 1), jnp.float32),
        scratch_shapes=[
            pltpu.VMEM((_C, _C), jnp.bfloat16),
            pltpu.VMEM((_C, 2), jnp.float32),
        ],
        compiler_params=pltpu.CompilerParams(
            dimension_semantics=("arbitrary",),
        ),
    )(g3, xt, mav_table, pc2)
    return jnp.reshape(out, ())


# speedup vs baseline: 7.0592x; 1.0334x over previous
import jax
import jax.numpy as jnp
from jax.experimental import pallas as pl
from jax.experimental.pallas import tpu as pltpu

_C = 19
_B = 65536
_EPS = 1e-30


def _owloss_tc_kernel(g_ref, x_ref, mav_ref, pc_ref, out_ref, wb_ref, acc):
    i = pl.program_id(0)
    nsteps = pl.num_programs(0)

    @pl.when(i == 0)
    def _init():
        acc[...] = jnp.zeros_like(acc)
        mav = mav_ref[...]
        mns = jnp.sum(mav * mav, axis=1, keepdims=True)
        w = mav * jax.lax.rsqrt(jnp.maximum(mns, _EPS))
        wb_ref[...] = w.astype(jnp.bfloat16)

    xt = x_ref[...]
    g = g_ref[0]

    at = jax.lax.dot_general(wb_ref[...], xt, (((1,), (0,)), ((), ())),
                             preferred_element_type=jnp.float32)
    ones_row = jnp.ones((1, _C), jnp.bfloat16)
    nsq = jax.lax.dot_general(ones_row, xt * xt, (((1,), (0,)), ((), ())),
                              preferred_element_type=jnp.float32)
    rnl = jax.lax.rsqrt(jnp.maximum(nsq, _EPS))

    lbl = jax.lax.broadcasted_iota(jnp.int32, (_C, 1), 0)
    msk = lbl == g
    num = jnp.sum(jnp.where(msk, at, 0.0), axis=0, keepdims=True)
    cos = (num * rnl).astype(jnp.bfloat16)
    ohb = msk.astype(jnp.bfloat16)
    cat = jnp.concatenate([cos, jnp.ones((1, _B), jnp.bfloat16)], axis=0)
    z = jax.lax.dot_general(ohb, cat, (((1,), (1,)), ((), ())),
                            preferred_element_type=jnp.float32)
    acc[...] += z

    @pl.when(i == nsteps - 1)
    def _finish():
        cs = acc[:, 0:1]
        c = acc[:, 1:2]
        pc = pc_ref[...]
        present = c > 0.0
        minl = jnp.min(jnp.where(present, lbl, _C))
        include = present & (lbl != minl) & (pc > 0.0)
        means = (c - cs) / jnp.maximum(c, 1.0)
        terms = jnp.where(include, means, 0.0)
        out_ref[...] = jnp.sum(terms, axis=(0, 1), keepdims=True).reshape(1, 1)


def kernel(logits, sem_gt, is_train, mav_table, prev_count):
    n = logits.shape[0]
    nsteps = n // _B
    xt = logits.astype(jnp.bfloat16).T
    g3 = sem_gt.reshape(nsteps, 1, _B)
    pc2 = prev_count.reshape(_C, 1)
    out = pl.pallas_call(
        _owloss_tc_kernel,
        grid=(nsteps,),
        in_specs=[
            pl.BlockSpec((1, 1, _B), lambda i: (i, 0, 0)),
            pl.BlockSpec((_C, _B), lambda i: (0, i)),
            pl.BlockSpec((_C, _C), lambda i: (0, 0)),
            pl.BlockSpec((_C, 1), lambda i: (0, 0)),
        ],
        out_specs=pl.BlockSpec((1, 1), lambda i: (0, 0)),
        out_shape=jax.ShapeDtypeStruct((1, 1), jnp.float32),
        scratch_shapes=[
            pltpu.VMEM((_C, _C), jnp.bfloat16),
            pltpu.VMEM((_C, 2), jnp.float32),
        ],
        compiler_params=pltpu.CompilerParams(
            dimension_semantics=("arbitrary",),
        ),
    )(g3, xt, mav_table, pc2)
    return jnp.reshape(out, ())
